# baseline (speedup 1.0000x reference)
import functools

import jax
import jax.numpy as jnp
import numpy as np
from jax import lax
from jax.experimental import pallas as pl
from jax.experimental.pallas import tpu as pltpu
from jax.experimental.pallas import tpu_sc as plsc

_DX = 768
_HW = 49
_F = 4096
_M = 16
_FB = 512
_RB = 384
_LANES = 128
_RUNLEN = _HW * _LANES

_SQRT_HALF = np.sqrt(0.5).astype(np.float32)

_ERFC_P = [2.326819970068386e-2, -1.387039388740657e-1, 3.687424674597105e-1,
           -5.824733027278666e-1, 6.210004621745983e-1, -4.944515323274145e-1,
           3.404879937665872e-1, -2.741127028184656e-1, 5.638259427386472e-1]
_ERFC_R = [-1.047766399936249e+1, 1.297719955372516e+1, -7.495518717768503e+0,
           2.921019019210786e+0, -1.015265279202700e+0, 4.218463358204948e-1,
           -2.820767439740514e-1, 5.641895067754075e-1]
_ERF_T = [7.853861353153693e-5, -8.010193625184903e-4, 5.188327685732524e-3,
          -2.685381193529856e-2, 1.128358514861418e-1, -3.761262582423300e-1,
          1.128379165726710e+0]


def _poly(y, coefs):
    p = jnp.full_like(y, np.float32(coefs[0]))
    for c in coefs[1:]:
        p = p * y + np.float32(c)
    return p


def _erfc32(x):
    abs_x = jnp.abs(x)
    z = jnp.exp(-x * x)
    q = 1.0 / abs_x
    y2 = q * q
    p = jnp.where(abs_x < 2.0, _poly(y2, _ERFC_P), _poly(y2, _ERFC_R))
    y = z * q * p
    big = jnp.where(x < 0.0, 2.0 - y, y)
    small = 1.0 - x * _poly(x * x, _ERF_T)
    return jnp.where(abs_x > 1.0, big, small)


def _gelu_exact(h):
    return 0.5 * h * _erfc32(-h * _SQRT_HALF)


def _pool_body(x_ref, o_ref):
    j = pl.program_id(1)
    plane = x_ref[0]

    @pl.when(j == 0)
    def _():
        o_ref[...] = plane

    @pl.when(j != 0)
    def _():
        o_ref[...] = o_ref[...] + plane

    @pl.when(j == _HW - 1)
    def _():
        o_ref[...] = o_ref[...] * (1.0 / _HW)


def _gate_body(p_ref, d_ref, w_ref, b_ref, wg_ref, logits_ref, acc_ref):
    f = pl.program_id(0)
    nf = pl.num_programs(0)
    cat = jnp.concatenate([p_ref[...], d_ref[...]], axis=1)
    h = jnp.dot(cat, w_ref[...], preferred_element_type=jnp.float32)
    h = h + b_ref[...]
    g = _gelu_exact(h)
    part = jnp.dot(g, wg_ref[...], preferred_element_type=jnp.float32)

    @pl.when(f == 0)
    def _():
        acc_ref[...] = part

    @pl.when(f != 0)
    def _():
        acc_ref[...] = acc_ref[...] + part

    @pl.when(f == nf - 1)
    def _():
        logits_ref[...] = acc_ref[...]


_NW = 32
_RPW = 1024 // _NW


def _route_sc_body(lg_hbm, gates_hbm, probs_hbm, idx_hbm,
                   lbuf, gbuf, pbuf, ibuf):
    wid = lax.axis_index("s") * 2 + lax.axis_index("c")
    base = wid * (_RPW * _M)
    pltpu.sync_copy(lg_hbm.at[pl.ds(base, _RPW * _M)], lbuf)
    iota = lax.iota(jnp.int32, _M)

    def _bfly(v, op):
        for k in (1, 2, 4, 8):
            v = op(v, v.at[iota ^ k].get(mode="promise_in_bounds"))
        return v

    for r in range(_RPW):
        v = lbuf[pl.ds(r * _M, _M)]
        m1 = _bfly(v, jnp.maximum)
        i1 = _bfly(jnp.where(v == m1, iota, _M), jnp.minimum)
        v2 = jnp.where(iota == i1, -jnp.inf, v)
        m2 = _bfly(v2, jnp.maximum)
        i2 = _bfly(jnp.where(v2 == m2, iota, _M), jnp.minimum)
        e = jnp.exp(v - m1)
        pbuf[pl.ds(r * _M, _M)] = e / _bfly(e, jnp.add)
        ev = jnp.exp(m2 - m1)
        g1 = 1.0 / (1.0 + ev)
        g2 = ev / (1.0 + ev)
        gbuf[pl.ds(r * _M, _M)] = (jnp.where(iota == i1, g1, 0.0)
                                   + jnp.where(iota == i2, g2, 0.0))
        ibuf[pl.ds(r * _M, _M)] = jnp.where(iota == 0, i1, i2)
    pltpu.sync_copy(gbuf, gates_hbm.at[pl.ds(base, _RPW * _M)])
    pltpu.sync_copy(pbuf, probs_hbm.at[pl.ds(base, _RPW * _M)])
    pltpu.sync_copy(ibuf, idx_hbm.at[pl.ds(base, _RPW * _M)])


_route_sc = functools.partial(
    pl.kernel,
    mesh=plsc.VectorSubcoreMesh(core_axis_name="c", subcore_axis_name="s"),
    out_type=[
        jax.ShapeDtypeStruct((1024 * _M,), jnp.float32),
        jax.ShapeDtypeStruct((1024 * _M,), jnp.float32),
        jax.ShapeDtypeStruct((1024 * _M,), jnp.int32),
    ],
    scratch_types=[
        pltpu.VMEM((_RPW * _M,), jnp.float32),
        pltpu.VMEM((_RPW * _M,), jnp.float32),
        pltpu.VMEM((_RPW * _M,), jnp.float32),
        pltpu.VMEM((_RPW * _M,), jnp.int32),
    ],
)(_route_sc_body)


def kernel(x, Degraded_feature, W_fusion, b_fusion, w_gate):
    B = x.shape[0]
    xt = jnp.transpose(x, (2, 3, 0, 1)).reshape(_HW, B, _DX)
    bb = 256
    pooled = pl.pallas_call(
        _pool_body,
        grid=(B // bb, _HW),
        in_specs=[pl.BlockSpec((1, bb, _DX), lambda i, j: (j, i, 0))],
        out_specs=pl.BlockSpec((bb, _DX), lambda i, j: (i, 0)),
        out_shape=jax.ShapeDtypeStruct((B, _DX), jnp.float32),
    )(xt)
    b2 = b_fusion.reshape(1, _F)
    dt = Degraded_feature.shape[1]
    logits = pl.pallas_call(
        _gate_body,
        grid=(_F // _FB,),
        in_specs=[
            pl.BlockSpec((B, _DX), lambda f: (0, 0)),
            pl.BlockSpec((B, dt), lambda f: (0, 0)),
            pl.BlockSpec((_DX + dt, _FB), lambda f: (0, f)),
            pl.BlockSpec((1, _FB), lambda f: (0, f)),
            pl.BlockSpec((_FB, _M), lambda f: (f, 0)),
        ],
        out_specs=pl.BlockSpec((B, _M), lambda f: (0, 0)),
        out_shape=jax.ShapeDtypeStruct((B, _M), jnp.float32),
        scratch_shapes=[pltpu.VMEM((B, _M), jnp.float32)],
    )(pooled, Degraded_feature, W_fusion, b2, w_gate)
    gates_f, probs_f, idx_f = _route_sc(logits.reshape(B * _M))
    gates = gates_f.reshape(B, _M)
    probs = probs_f.reshape(B, _M)
    idx = idx_f.reshape(B, _M)[:, :2]
    moe_loss = jnp.zeros((), jnp.float32)
    return (gates, moe_loss, probs, idx)


# speedup vs baseline: 1.6481x; 1.6481x over previous
import functools

import jax
import jax.numpy as jnp
import numpy as np
from jax import lax
from jax.experimental import pallas as pl
from jax.experimental.pallas import tpu as pltpu
from jax.experimental.pallas import tpu_sc as plsc

_DX = 768
_HW = 49
_F = 4096
_M = 16
_FB = 512
_RB = 384
_LANES = 128
_RUNLEN = _HW * _LANES

_SQRT_HALF = np.sqrt(0.5).astype(np.float32)

_ERFC_P = [2.326819970068386e-2, -1.387039388740657e-1, 3.687424674597105e-1,
           -5.824733027278666e-1, 6.210004621745983e-1, -4.944515323274145e-1,
           3.404879937665872e-1, -2.741127028184656e-1, 5.638259427386472e-1]
_ERFC_R = [-1.047766399936249e+1, 1.297719955372516e+1, -7.495518717768503e+0,
           2.921019019210786e+0, -1.015265279202700e+0, 4.218463358204948e-1,
           -2.820767439740514e-1, 5.641895067754075e-1]
_ERF_T = [7.853861353153693e-5, -8.010193625184903e-4, 5.188327685732524e-3,
          -2.685381193529856e-2, 1.128358514861418e-1, -3.761262582423300e-1,
          1.128379165726710e+0]


def _poly(y, coefs):
    p = jnp.full_like(y, np.float32(coefs[0]))
    for c in coefs[1:]:
        p = p * y + np.float32(c)
    return p


def _erfc32(x):
    abs_x = jnp.abs(x)
    z = jnp.exp(-x * x)
    q = 1.0 / abs_x
    y2 = q * q
    p = jnp.where(abs_x < 2.0, _poly(y2, _ERFC_P), _poly(y2, _ERFC_R))
    y = z * q * p
    big = jnp.where(x < 0.0, 2.0 - y, y)
    small = 1.0 - x * _poly(x * x, _ERF_T)
    return jnp.where(abs_x > 1.0, big, small)


def _gelu_exact(h):
    return 0.5 * h * _erfc32(-h * _SQRT_HALF)


def _pool_body(x_ref, o_ref):
    j = pl.program_id(1)

    @pl.when(j == 0)
    def _():
        acc = x_ref[0]
        for i in range(1, 7):
            acc = acc + x_ref[i]
        o_ref[...] = acc

    @pl.when(j != 0)
    def _():
        acc = o_ref[...]
        for i in range(7):
            acc = acc + x_ref[i]
        o_ref[...] = acc

    @pl.when(j == 6)
    def _():
        o_ref[...] = o_ref[...] * (1.0 / _HW)


def _gate_body(p_ref, d_ref, w_ref, b_ref, wg_ref, logits_ref, acc_ref):
    f = pl.program_id(0)
    nf = pl.num_programs(0)
    cat = jnp.concatenate([p_ref[...], d_ref[...]], axis=1)
    h = jnp.dot(cat, w_ref[...], preferred_element_type=jnp.float32)
    h = h + b_ref[...]
    g = _gelu_exact(h)
    part = jnp.dot(g, wg_ref[...], preferred_element_type=jnp.float32)

    @pl.when(f == 0)
    def _():
        acc_ref[...] = part

    @pl.when(f != 0)
    def _():
        acc_ref[...] = acc_ref[...] + part

    @pl.when(f == nf - 1)
    def _():
        logits_ref[...] = acc_ref[...]


_NW = 32
_RPW = 1024 // _NW


def _route_sc_body(lg_hbm, gates_hbm, probs_hbm, idx_hbm,
                   lbuf, gbuf, pbuf, ibuf):
    wid = lax.axis_index("s") * 2 + lax.axis_index("c")
    base = wid * (_RPW * _M)
    pltpu.sync_copy(lg_hbm.at[pl.ds(base, _RPW * _M)], lbuf)
    iota = lax.iota(jnp.int32, _M)

    def _bfly(v, op):
        for k in (1, 2, 4, 8):
            v = op(v, v.at[iota ^ k].get(mode="promise_in_bounds"))
        return v

    for r in range(_RPW):
        v = lbuf[pl.ds(r * _M, _M)]
        m1 = _bfly(v, jnp.maximum)
        i1 = _bfly(jnp.where(v == m1, iota, _M), jnp.minimum)
        v2 = jnp.where(iota == i1, -jnp.inf, v)
        m2 = _bfly(v2, jnp.maximum)
        i2 = _bfly(jnp.where(v2 == m2, iota, _M), jnp.minimum)
        e = jnp.exp(v - m1)
        pbuf[pl.ds(r * _M, _M)] = e / _bfly(e, jnp.add)
        ev = jnp.exp(m2 - m1)
        g1 = 1.0 / (1.0 + ev)
        g2 = ev / (1.0 + ev)
        gbuf[pl.ds(r * _M, _M)] = (jnp.where(iota == i1, g1, 0.0)
                                   + jnp.where(iota == i2, g2, 0.0))
        ibuf[pl.ds(r * _M, _M)] = jnp.where(iota == 0, i1, i2)
    pltpu.sync_copy(gbuf, gates_hbm.at[pl.ds(base, _RPW * _M)])
    pltpu.sync_copy(pbuf, probs_hbm.at[pl.ds(base, _RPW * _M)])
    pltpu.sync_copy(ibuf, idx_hbm.at[pl.ds(base, _RPW * _M)])


_route_sc = functools.partial(
    pl.kernel,
    mesh=plsc.VectorSubcoreMesh(core_axis_name="c", subcore_axis_name="s"),
    out_type=[
        jax.ShapeDtypeStruct((1024 * _M,), jnp.float32),
        jax.ShapeDtypeStruct((1024 * _M,), jnp.float32),
        jax.ShapeDtypeStruct((1024 * _M,), jnp.int32),
    ],
    scratch_types=[
        pltpu.VMEM((_RPW * _M,), jnp.float32),
        pltpu.VMEM((_RPW * _M,), jnp.float32),
        pltpu.VMEM((_RPW * _M,), jnp.float32),
        pltpu.VMEM((_RPW * _M,), jnp.int32),
    ],
)(_route_sc_body)


def kernel(x, Degraded_feature, W_fusion, b_fusion, w_gate):
    B = x.shape[0]
    xt = jnp.transpose(x, (2, 3, 0, 1)).reshape(_HW, B, _DX)
    bb = 512
    pooled = pl.pallas_call(
        _pool_body,
        grid=(B // bb, 7),
        in_specs=[pl.BlockSpec((7, bb, _DX), lambda i, j: (j, i, 0))],
        out_specs=pl.BlockSpec((bb, _DX), lambda i, j: (i, 0)),
        out_shape=jax.ShapeDtypeStruct((B, _DX), jnp.float32),
    )(xt)
    b2 = b_fusion.reshape(1, _F)
    dt = Degraded_feature.shape[1]
    logits = pl.pallas_call(
        _gate_body,
        grid=(_F // _FB,),
        in_specs=[
            pl.BlockSpec((B, _DX), lambda f: (0, 0)),
            pl.BlockSpec((B, dt), lambda f: (0, 0)),
            pl.BlockSpec((_DX + dt, _FB), lambda f: (0, f)),
            pl.BlockSpec((1, _FB), lambda f: (0, f)),
            pl.BlockSpec((_FB, _M), lambda f: (f, 0)),
        ],
        out_specs=pl.BlockSpec((B, _M), lambda f: (0, 0)),
        out_shape=jax.ShapeDtypeStruct((B, _M), jnp.float32),
        scratch_shapes=[pltpu.VMEM((B, _M), jnp.float32)],
    )(pooled, Degraded_feature, W_fusion, b2, w_gate)
    gates_f, probs_f, idx_f = _route_sc(logits.reshape(B * _M))
    gates = gates_f.reshape(B, _M)
    probs = probs_f.reshape(B, _M)
    idx = idx_f.reshape(B, _M)[:, :2]
    moe_loss = jnp.zeros((), jnp.float32)
    return (gates, moe_loss, probs, idx)


# speedup vs baseline: 1.6742x; 1.0158x over previous
import functools

import jax
import jax.numpy as jnp
import numpy as np
from jax import lax
from jax.experimental import pallas as pl
from jax.experimental.pallas import tpu as pltpu
from jax.experimental.pallas import tpu_sc as plsc

_DX = 768
_HW = 49
_F = 4096
_M = 16
_FB = 1024
_RB = 384
_LANES = 128
_RUNLEN = _HW * _LANES

_SQRT_HALF = np.sqrt(0.5).astype(np.float32)

_ERFC_P = [2.326819970068386e-2, -1.387039388740657e-1, 3.687424674597105e-1,
           -5.824733027278666e-1, 6.210004621745983e-1, -4.944515323274145e-1,
           3.404879937665872e-1, -2.741127028184656e-1, 5.638259427386472e-1]
_ERFC_R = [-1.047766399936249e+1, 1.297719955372516e+1, -7.495518717768503e+0,
           2.921019019210786e+0, -1.015265279202700e+0, 4.218463358204948e-1,
           -2.820767439740514e-1, 5.641895067754075e-1]
_ERF_T = [7.853861353153693e-5, -8.010193625184903e-4, 5.188327685732524e-3,
          -2.685381193529856e-2, 1.128358514861418e-1, -3.761262582423300e-1,
          1.128379165726710e+0]


def _poly(y, coefs):
    p = jnp.full_like(y, np.float32(coefs[0]))
    for c in coefs[1:]:
        p = p * y + np.float32(c)
    return p


def _erfc32(x):
    abs_x = jnp.abs(x)
    z = jnp.exp(-x * x)
    q = 1.0 / abs_x
    y2 = q * q
    p = jnp.where(abs_x < 2.0, _poly(y2, _ERFC_P), _poly(y2, _ERFC_R))
    y = z * q * p
    big = jnp.where(x < 0.0, 2.0 - y, y)
    small = 1.0 - x * _poly(x * x, _ERF_T)
    return jnp.where(abs_x > 1.0, big, small)


def _gelu_exact(h):
    return 0.5 * h * _erfc32(-h * _SQRT_HALF)


def _pool_body(x_ref, o_ref):
    j = pl.program_id(1)

    @pl.when(j == 0)
    def _():
        acc = x_ref[0]
        for i in range(1, 7):
            acc = acc + x_ref[i]
        o_ref[...] = acc

    @pl.when(j != 0)
    def _():
        acc = o_ref[...]
        for i in range(7):
            acc = acc + x_ref[i]
        o_ref[...] = acc

    @pl.when(j == 6)
    def _():
        o_ref[...] = o_ref[...] * (1.0 / _HW)


def _gate_body(p_ref, d_ref, w_ref, b_ref, wg_ref, logits_ref, acc_ref):
    f = pl.program_id(0)
    nf = pl.num_programs(0)
    cat = jnp.concatenate([p_ref[...], d_ref[...]], axis=1)
    h = jnp.dot(cat, w_ref[...], preferred_element_type=jnp.float32)
    h = h + b_ref[...]
    g = _gelu_exact(h)
    part = jnp.dot(g, wg_ref[...], preferred_element_type=jnp.float32)

    @pl.when(f == 0)
    def _():
        acc_ref[...] = part

    @pl.when(f != 0)
    def _():
        acc_ref[...] = acc_ref[...] + part

    @pl.when(f == nf - 1)
    def _():
        logits_ref[...] = acc_ref[...]


_NW = 32
_RPW = 1024 // _NW


def _route_sc_body(lg_hbm, gates_hbm, probs_hbm, idx_hbm,
                   lbuf, gbuf, pbuf, ibuf):
    wid = lax.axis_index("s") * 2 + lax.axis_index("c")
    base = wid * (_RPW * _M)
    pltpu.sync_copy(lg_hbm.at[pl.ds(base, _RPW * _M)], lbuf)
    iota = lax.iota(jnp.int32, _M)

    def _bfly(v, op):
        for k in (1, 2, 4, 8):
            v = op(v, v.at[iota ^ k].get(mode="promise_in_bounds"))
        return v

    for r in range(_RPW):
        v = lbuf[pl.ds(r * _M, _M)]
        m1 = _bfly(v, jnp.maximum)
        i1 = _bfly(jnp.where(v == m1, iota, _M), jnp.minimum)
        v2 = jnp.where(iota == i1, -jnp.inf, v)
        m2 = _bfly(v2, jnp.maximum)
        i2 = _bfly(jnp.where(v2 == m2, iota, _M), jnp.minimum)
        e = jnp.exp(v - m1)
        pbuf[pl.ds(r * _M, _M)] = e / _bfly(e, jnp.add)
        ev = jnp.exp(m2 - m1)
        g1 = 1.0 / (1.0 + ev)
        g2 = ev / (1.0 + ev)
        gbuf[pl.ds(r * _M, _M)] = (jnp.where(iota == i1, g1, 0.0)
                                   + jnp.where(iota == i2, g2, 0.0))
        ibuf[pl.ds(r * _M, _M)] = jnp.where(iota == 0, i1, i2)
    pltpu.sync_copy(gbuf, gates_hbm.at[pl.ds(base, _RPW * _M)])
    pltpu.sync_copy(pbuf, probs_hbm.at[pl.ds(base, _RPW * _M)])
    pltpu.sync_copy(ibuf, idx_hbm.at[pl.ds(base, _RPW * _M)])


_route_sc = functools.partial(
    pl.kernel,
    mesh=plsc.VectorSubcoreMesh(core_axis_name="c", subcore_axis_name="s"),
    out_type=[
        jax.ShapeDtypeStruct((1024 * _M,), jnp.float32),
        jax.ShapeDtypeStruct((1024 * _M,), jnp.float32),
        jax.ShapeDtypeStruct((1024 * _M,), jnp.int32),
    ],
    scratch_types=[
        pltpu.VMEM((_RPW * _M,), jnp.float32),
        pltpu.VMEM((_RPW * _M,), jnp.float32),
        pltpu.VMEM((_RPW * _M,), jnp.float32),
        pltpu.VMEM((_RPW * _M,), jnp.int32),
    ],
)(_route_sc_body)


def kernel(x, Degraded_feature, W_fusion, b_fusion, w_gate):
    B = x.shape[0]
    xt = jnp.transpose(x, (2, 3, 0, 1)).reshape(_HW, B, _DX)
    bb = 1024
    pooled = pl.pallas_call(
        _pool_body,
        grid=(B // bb, 7),
        in_specs=[pl.BlockSpec((7, bb, _DX), lambda i, j: (j, i, 0))],
        out_specs=pl.BlockSpec((bb, _DX), lambda i, j: (i, 0)),
        out_shape=jax.ShapeDtypeStruct((B, _DX), jnp.float32),
    )(xt)
    b2 = b_fusion.reshape(1, _F)
    dt = Degraded_feature.shape[1]
    logits = pl.pallas_call(
        _gate_body,
        grid=(_F // _FB,),
        in_specs=[
            pl.BlockSpec((B, _DX), lambda f: (0, 0)),
            pl.BlockSpec((B, dt), lambda f: (0, 0)),
            pl.BlockSpec((_DX + dt, _FB), lambda f: (0, f)),
            pl.BlockSpec((1, _FB), lambda f: (0, f)),
            pl.BlockSpec((_FB, _M), lambda f: (f, 0)),
        ],
        out_specs=pl.BlockSpec((B, _M), lambda f: (0, 0)),
        out_shape=jax.ShapeDtypeStruct((B, _M), jnp.float32),
        scratch_shapes=[pltpu.VMEM((B, _M), jnp.float32)],
    )(pooled, Degraded_feature, W_fusion, b2, w_gate)
    gates_f, probs_f, idx_f = _route_sc(logits.reshape(B * _M))
    gates = gates_f.reshape(B, _M)
    probs = probs_f.reshape(B, _M)
    idx = idx_f.reshape(B, _M)[:, :2]
    moe_loss = jnp.zeros((), jnp.float32)
    return (gates, moe_loss, probs, idx)


# speedup vs baseline: 1.6814x; 1.0043x over previous
import functools

import jax
import jax.numpy as jnp
import numpy as np
from jax import lax
from jax.experimental import pallas as pl
from jax.experimental.pallas import tpu as pltpu
from jax.experimental.pallas import tpu_sc as plsc

_DX = 768
_HW = 49
_F = 4096
_M = 16
_FB = 1024

_SQRT_HALF = np.sqrt(0.5).astype(np.float32)

_ERFC_P = [2.326819970068386e-2, -1.387039388740657e-1, 3.687424674597105e-1,
           -5.824733027278666e-1, 6.210004621745983e-1, -4.944515323274145e-1,
           3.404879937665872e-1, -2.741127028184656e-1, 5.638259427386472e-1]
_ERFC_R = [-1.047766399936249e+1, 1.297719955372516e+1, -7.495518717768503e+0,
           2.921019019210786e+0, -1.015265279202700e+0, 4.218463358204948e-1,
           -2.820767439740514e-1, 5.641895067754075e-1]
_ERF_T = [7.853861353153693e-5, -8.010193625184903e-4, 5.188327685732524e-3,
          -2.685381193529856e-2, 1.128358514861418e-1, -3.761262582423300e-1,
          1.128379165726710e+0]


def _poly(y, coefs):
    p = jnp.full_like(y, np.float32(coefs[0]))
    for c in coefs[1:]:
        p = p * y + np.float32(c)
    return p


def _erfc32(x):
    abs_x = jnp.abs(x)
    z = jnp.exp(-x * x)
    q = 1.0 / abs_x
    y2 = q * q
    p = jnp.where(abs_x < 2.0, _poly(y2, _ERFC_P), _poly(y2, _ERFC_R))
    y = z * q * p
    big = jnp.where(x < 0.0, 2.0 - y, y)
    small = 1.0 - x * _poly(x * x, _ERF_T)
    return jnp.where(abs_x > 1.0, big, small)


def _gelu_exact(h):
    return 0.5 * h * _erfc32(-h * _SQRT_HALF)


def _pool_body(x_ref, o_ref):
    j = pl.program_id(1)

    @pl.when(j == 0)
    def _():
        acc = x_ref[0]
        for i in range(1, 7):
            acc = acc + x_ref[i]
        o_ref[...] = acc

    @pl.when(j != 0)
    def _():
        acc = o_ref[...]
        for i in range(7):
            acc = acc + x_ref[i]
        o_ref[...] = acc

    @pl.when(j == 6)
    def _():
        o_ref[...] = o_ref[...] * (1.0 / _HW)


def _gate_body(p_ref, d_ref, w_ref, b_ref, wg_ref, logits_ref, acc_ref):
    f = pl.program_id(0)
    nf = pl.num_programs(0)
    cat = jnp.concatenate([p_ref[...], d_ref[...]], axis=1)
    h = jnp.dot(cat, w_ref[...], preferred_element_type=jnp.float32)
    h = h + b_ref[...]
    g = _gelu_exact(h)
    part = jnp.dot(g, wg_ref[...], preferred_element_type=jnp.float32)

    @pl.when(f == 0)
    def _():
        acc_ref[...] = part

    @pl.when(f != 0)
    def _():
        acc_ref[...] = acc_ref[...] + part

    @pl.when(f == nf - 1)
    def _():
        logits_ref[...] = acc_ref[...]


_NW = 32
_RPW = 1024 // _NW


def _route_sc_body(lg_hbm, gates_hbm, probs_hbm, idx_hbm,
                   lbuf, gbuf, pbuf, ibuf):
    wid = lax.axis_index("s") * 2 + lax.axis_index("c")
    base = wid * (_RPW * _M)
    pltpu.sync_copy(lg_hbm.at[pl.ds(base, _RPW * _M)], lbuf)
    iota = lax.iota(jnp.int32, _M)

    def _bfly(v, op):
        for k in (1, 2, 4, 8):
            v = op(v, v.at[iota ^ k].get(mode="promise_in_bounds"))
        return v

    for r in range(_RPW):
        v = lbuf[pl.ds(r * _M, _M)]
        m1 = _bfly(v, jnp.maximum)
        i1 = _bfly(jnp.where(v == m1, iota, _M), jnp.minimum)
        v2 = jnp.where(iota == i1, -jnp.inf, v)
        m2 = _bfly(v2, jnp.maximum)
        i2 = _bfly(jnp.where(v2 == m2, iota, _M), jnp.minimum)
        e = jnp.exp(v - m1)
        pbuf[pl.ds(r * _M, _M)] = e / _bfly(e, jnp.add)
        ev = jnp.exp(m2 - m1)
        g1 = 1.0 / (1.0 + ev)
        g2 = ev / (1.0 + ev)
        gbuf[pl.ds(r * _M, _M)] = (jnp.where(iota == i1, g1, 0.0)
                                   + jnp.where(iota == i2, g2, 0.0))
        ibuf[pl.ds(r * _M, _M)] = jnp.where(iota == 0, i1, i2)
    pltpu.sync_copy(gbuf, gates_hbm.at[pl.ds(base, _RPW * _M)])
    pltpu.sync_copy(pbuf, probs_hbm.at[pl.ds(base, _RPW * _M)])
    pltpu.sync_copy(ibuf, idx_hbm.at[pl.ds(base, _RPW * _M)])


_route_sc = functools.partial(
    pl.kernel,
    mesh=plsc.VectorSubcoreMesh(core_axis_name="c", subcore_axis_name="s"),
    out_type=[
        jax.ShapeDtypeStruct((1024 * _M,), jnp.float32),
        jax.ShapeDtypeStruct((1024 * _M,), jnp.float32),
        jax.ShapeDtypeStruct((1024 * _M,), jnp.int32),
    ],
    scratch_types=[
        pltpu.VMEM((_RPW * _M,), jnp.float32),
        pltpu.VMEM((_RPW * _M,), jnp.float32),
        pltpu.VMEM((_RPW * _M,), jnp.float32),
        pltpu.VMEM((_RPW * _M,), jnp.int32),
    ],
)(_route_sc_body)


def kernel(x, Degraded_feature, W_fusion, b_fusion, w_gate):
    B = x.shape[0]
    xt = jnp.transpose(x, (2, 3, 0, 1)).reshape(_HW, B, _DX)
    bb = 1024
    pooled = pl.pallas_call(
        _pool_body,
        grid=(B // bb, 7),
        in_specs=[pl.BlockSpec((7, bb, _DX), lambda i, j: (j, i, 0))],
        out_specs=pl.BlockSpec((bb, _DX), lambda i, j: (i, 0)),
        out_shape=jax.ShapeDtypeStruct((B, _DX), jnp.float32),
    )(xt)
    b2 = b_fusion.reshape(1, _F)
    dt = Degraded_feature.shape[1]
    logits = pl.pallas_call(
        _gate_body,
        grid=(_F // _FB,),
        in_specs=[
            pl.BlockSpec((B, _DX), lambda f: (0, 0)),
            pl.BlockSpec((B, dt), lambda f: (0, 0)),
            pl.BlockSpec((_DX + dt, _FB), lambda f: (0, f)),
            pl.BlockSpec((1, _FB), lambda f: (0, f)),
            pl.BlockSpec((_FB, _M), lambda f: (f, 0)),
        ],
        out_specs=pl.BlockSpec((B, _M), lambda f: (0, 0)),
        out_shape=jax.ShapeDtypeStruct((B, _M), jnp.float32),
        scratch_shapes=[pltpu.VMEM((B, _M), jnp.float32)],
    )(pooled, Degraded_feature, W_fusion, b2, w_gate)
    gates_f, probs_f, idx_f = _route_sc(logits.reshape(B * _M))
    gates = gates_f.reshape(B, _M)
    probs = probs_f.reshape(B, _M)
    idx = idx_f.reshape(B, _M)[:, :2]
    moe_loss = jnp.zeros((), jnp.float32)
    return (gates, moe_loss, probs, idx)
